# baseline (speedup 1.0000x reference)
import functools

import jax
import jax.numpy as jnp
from jax import lax
from jax.experimental import pallas as pl
from jax.experimental.pallas import tpu as pltpu
from jax.experimental.pallas import tpu_sc as plsc

_NC = 2
_NS = 16
_CHUNK = 128
_NB = 2
_NI = 4


def _make_sc_aggregate(N, D, E, with_count):
    NW = _NC * _NS
    assert E % _CHUNK == 0
    NCH = E // _CHUNK
    G = NCH // NW
    xrem = NCH - G * NW
    assert G >= 2 * _NI and _NI == 2 * _NB
    rpt = (N // _NS) // 8 * 8
    rrem = N - _NS * rpt
    assert rrem % 8 == 0 and N % 16 == 0

    out_type = [jax.ShapeDtypeStruct((N, D), jnp.float32),
                jax.ShapeDtypeStruct((N, D), jnp.float32)]
    if with_count:
        out_type += [jax.ShapeDtypeStruct((N,), jnp.float32),
                     jax.ShapeDtypeStruct((N,), jnp.float32)]

    scratch = (
        [pltpu.VMEM((2, _CHUNK), jnp.int32) for _ in range(_NI)]
        + [pltpu.VMEM((_CHUNK, D), jnp.float32) for _ in range(_NB)]
        + [pltpu.VMEM_SHARED((N, D), jnp.float32)]
        + [pltpu.SemaphoreType.DMA for _ in range(_NI)]
        + [pltpu.SemaphoreType.DMA for _ in range(_NB)]
        + [pltpu.SemaphoreType.DMA for _ in range(_NI)]
    )
    if with_count:
        scratch += [
            pltpu.VMEM((_CHUNK,), jnp.float32),
            pltpu.VMEM_SHARED((N,), jnp.float32),
            pltpu.VMEM((rpt,), jnp.float32),
        ]

    def body(*refs):
        it = iter(refs)
        edge_ref, feat_ref = next(it), next(it)
        psum0_ref, psum1_ref = next(it), next(it)
        if with_count:
            cnt0_ref, cnt1_ref = next(it), next(it)
        idx_v = [next(it) for _ in range(_NI)]
        rows_v = [next(it) for _ in range(_NB)]
        acc_sh = next(it)
        isem = [next(it) for _ in range(_NI)]
        gsem = [next(it) for _ in range(_NB)]
        ssem = [next(it) for _ in range(_NI)]
        if with_count:
            ones_v, cnt_sh, cbounce_v = next(it), next(it), next(it)

        c = lax.axis_index("c")
        s = lax.axis_index("s")
        wid = c * _NS + s
        r0 = s * rpt

        def zfill(i, carry):
            r = i // (D // 16)
            o = (i % (D // 16)) * 16
            rows_v[0][r, pl.ds(o, 16)] = jnp.zeros((16,), jnp.float32)
            return carry
        lax.fori_loop(0, _CHUNK * (D // 16), zfill, 0)
        nrep = rpt // _CHUNK
        for k in range(nrep):
            pltpu.sync_copy(rows_v[0],
                            acc_sh.at[pl.ds(r0 + k * _CHUNK, _CHUNK)])
        zr = rpt - nrep * _CHUNK
        if zr:
            pltpu.sync_copy(rows_v[0].at[pl.ds(0, zr)],
                            acc_sh.at[pl.ds(r0 + nrep * _CHUNK, zr)])
        if with_count:
            def zero16(i, carry):
                cbounce_v[pl.ds(i * 16, 16)] = jnp.zeros((16,), jnp.float32)
                return carry
            lax.fori_loop(0, rpt // 16, zero16, 0)
            pltpu.sync_copy(cbounce_v, cnt_sh.at[pl.ds(r0, rpt)])
            for i in range(_CHUNK // 16):
                ones_v[pl.ds(i * 16, 16)] = jnp.ones((16,), jnp.float32)
        if rrem:
            @pl.when(s == _NS - 1)
            def _():
                rr = _NS * rpt
                pltpu.sync_copy(rows_v[0].at[pl.ds(0, rrem)],
                                acc_sh.at[pl.ds(rr, rrem)])
                if with_count:
                    pltpu.sync_copy(cbounce_v.at[pl.ds(0, rrem)],
                                    cnt_sh.at[pl.ds(rr, rrem)])
        plsc.subcore_barrier()

        def echunk(g):
            return jnp.minimum(wid + g * NW, NCH - 1) * _CHUNK

        def idx_load(g, b):
            eb = echunk(g)
            pltpu.async_copy(edge_ref.at[:, pl.ds(eb, _CHUNK)],
                             idx_v[b], isem[b])

        def idx_wait(g, b):
            eb = echunk(g)
            pltpu.make_async_copy(edge_ref.at[:, pl.ds(eb, _CHUNK)],
                                  idx_v[b], isem[b]).wait()

        def gather_start(b2, b4):
            pltpu.async_copy(feat_ref.at[idx_v[b4].at[0]], rows_v[b2],
                             gsem[b2])

        def gather_wait(b2, b4):
            pltpu.make_async_copy(feat_ref.at[idx_v[b4].at[0]], rows_v[b2],
                                  gsem[b2]).wait()

        def scatter_start(b2, b4):
            pltpu.async_copy(rows_v[b2], acc_sh.at[idx_v[b4].at[1]],
                             ssem[b4], add=True)
            if with_count:
                pltpu.async_copy(ones_v, cnt_sh.at[idx_v[b4].at[1]],
                                 ssem[b4], add=True)

        def scatter_wait(b2, b4):
            pltpu.make_async_copy(rows_v[b2], acc_sh.at[idx_v[b4].at[1]],
                                  ssem[b4]).wait()
            if with_count:
                pltpu.make_async_copy(ones_v, cnt_sh.at[idx_v[b4].at[1]],
                                      ssem[b4]).wait()

        def steady(g, gb):
            b2, b4 = gb % _NB, gb % _NI
            p2, p4 = (gb - 1) % _NB, (gb - 1) % _NI
            idx_wait(g, b4)
            scatter_wait(b2, (gb - 2) % _NI)
            gather_start(b2, b4)
            gather_wait(p2, p4)
            scatter_start(p2, p4)
            idx_load(g + 2, (gb + 2) % _NI)

        idx_load(0, 0)
        idx_load(1, 1)
        idx_wait(0, 0)
        gather_start(0, 0)
        idx_load(2, 2)
        idx_wait(1, 1)
        gather_start(1, 1)
        gather_wait(0, 0)
        scatter_start(0, 0)
        idx_load(3, 3)

        n_steady = G - 2
        nq = n_steady // _NI

        def quad(q, carry):
            for r in range(_NI):
                steady(q * _NI + 2 + r, 2 + r)
            return carry

        lax.fori_loop(0, nq, quad, 0)
        for g in range(nq * _NI + 2, G):
            steady(g, g)

        gather_wait((G - 1) % _NB, (G - 1) % _NI)
        scatter_start((G - 1) % _NB, (G - 1) % _NI)
        idx_wait(G, G % _NI)
        idx_wait(G + 1, (G + 1) % _NI)
        scatter_wait(G % _NB, (G - 2) % _NI)
        if xrem:
            @pl.when(wid < xrem)
            def _():
                gather_start(G % _NB, G % _NI)
                gather_wait(G % _NB, G % _NI)
                scatter_start(G % _NB, G % _NI)
                scatter_wait(G % _NB, G % _NI)
        scatter_wait((G - 1) % _NB, (G - 1) % _NI)

        plsc.subcore_barrier()

        def flush(psum_ref, cnt_ref):
            pltpu.sync_copy(acc_sh.at[pl.ds(r0, rpt)],
                            psum_ref.at[pl.ds(r0, rpt)])
            if with_count:
                pltpu.sync_copy(cnt_sh.at[pl.ds(r0, rpt)], cbounce_v)
                pltpu.sync_copy(cbounce_v, cnt_ref.at[pl.ds(r0, rpt)])
            if rrem:
                @pl.when(s == _NS - 1)
                def _():
                    rr = _NS * rpt
                    pltpu.sync_copy(acc_sh.at[pl.ds(rr, rrem)],
                                    psum_ref.at[pl.ds(rr, rrem)])
                    if with_count:
                        pltpu.sync_copy(cnt_sh.at[pl.ds(rr, rrem)],
                                        cbounce_v.at[pl.ds(0, rrem)])
                        pltpu.sync_copy(cbounce_v.at[pl.ds(0, rrem)],
                                        cnt_ref.at[pl.ds(rr, rrem)])

        @pl.when(c == 0)
        def _():
            flush(psum0_ref, cnt0_ref if with_count else None)

        @pl.when(c == 1)
        def _():
            flush(psum1_ref, cnt1_ref if with_count else None)

    return pl.kernel(
        body,
        out_type=out_type,
        mesh=plsc.VectorSubcoreMesh(core_axis_name="c", subcore_axis_name="s"),
        scratch_types=scratch,
    )


def _tc_layer(N, D, H, relu, RB=5000):
    grid = (N // RB,)
    row_spec = pl.BlockSpec((RB, D), lambda i: (i, 0))
    col_spec = pl.BlockSpec((RB, 1), lambda i: (i, 0))
    w_spec = pl.BlockSpec((D, H), lambda i: (0, 0))
    b_spec = pl.BlockSpec((1, H), lambda i: (0, 0))

    def body(p0_ref, p1_ref, cc_ref, x_ref, wn_ref, ws_ref,
             b_ref, out_ref):
        cnt = cc_ref[:, 0:1] + cc_ref[:, 1:2]
        inv = 1.0 / jnp.maximum(cnt, 1.0)
        agg = (p0_ref[...] + p1_ref[...]) * inv
        acc = (jnp.dot(agg, wn_ref[...],
                       preferred_element_type=jnp.float32)
               + jnp.dot(x_ref[...], ws_ref[...],
                         preferred_element_type=jnp.float32)
               + b_ref[...])
        out_ref[...] = jnp.maximum(acc, 0.0) if relu else acc

    return pl.pallas_call(
        body,
        grid=grid,
        in_specs=[row_spec, row_spec,
                  pl.BlockSpec((RB, 2), lambda i: (i, 0)), row_spec,
                  w_spec, w_spec, b_spec],
        out_specs=pl.BlockSpec((RB, H), lambda i: (i, 0)),
        out_shape=jax.ShapeDtypeStruct((N, H), jnp.float32),
    )


def kernel(x, edge_index, W_neigh1, W_self1, b1, W_neigh2, W_self2, b2):
    N, D = x.shape
    H = W_neigh1.shape[1]
    E = edge_index.shape[1]

    p0, p1, c0, c1 = _make_sc_aggregate(N, D, E, with_count=True)(
        edge_index, x)
    cc = jnp.stack([c0, c1], axis=-1)
    h = _tc_layer(N, D, H, relu=True)(
        p0, p1, cc, x, W_neigh1, W_self1, b1.reshape(1, H))

    q0, q1 = _make_sc_aggregate(N, H, E, with_count=False)(edge_index, h)
    out = _tc_layer(N, H, H, relu=False)(
        q0, q1, cc, h, W_neigh2, W_self2, b2.reshape(1, H))
    return out


# speedup vs baseline: 1.0003x; 1.0003x over previous
import jax
import jax.numpy as jnp
from jax import lax
from jax.experimental import pallas as pl
from jax.experimental.pallas import tpu as pltpu
from jax.experimental.pallas import tpu_sc as plsc

_NC = 2
_NS = 16
_CHUNK = 128
_NB = 2
_NI = 4


def _make_sc_aggregate(N, D, E, with_count):
    NW = _NC * _NS
    assert E % _CHUNK == 0
    NCH = E // _CHUNK
    G = NCH // NW
    xrem = NCH - G * NW
    assert G >= 2 * _NI and _NI == 2 * _NB
    rpt = (N // _NS) // 8 * 8
    rrem = N - _NS * rpt
    assert rrem % 8 == 0 and N % 16 == 0

    out_type = [jax.ShapeDtypeStruct((N, D), jnp.float32),
                jax.ShapeDtypeStruct((N, D), jnp.float32)]
    if with_count:
        out_type += [jax.ShapeDtypeStruct((N,), jnp.float32),
                     jax.ShapeDtypeStruct((N,), jnp.float32)]

    scratch = (
        [pltpu.VMEM((2, _CHUNK), jnp.int32) for _ in range(_NI)]
        + [pltpu.VMEM((_CHUNK, D), jnp.float32) for _ in range(_NB)]
        + [pltpu.VMEM_SHARED((N, D), jnp.float32)]
        + [pltpu.SemaphoreType.DMA for _ in range(_NI)]
        + [pltpu.SemaphoreType.DMA for _ in range(_NB)]
        + [pltpu.SemaphoreType.DMA for _ in range(_NI)]
    )
    if with_count:
        scratch += [
            pltpu.VMEM((_CHUNK,), jnp.float32),
            pltpu.VMEM_SHARED((N,), jnp.float32),
            pltpu.VMEM((rpt,), jnp.float32),
        ]

    def body(*refs):
        it = iter(refs)
        edge_ref, feat_ref = next(it), next(it)
        psum0_ref, psum1_ref = next(it), next(it)
        if with_count:
            cnt0_ref, cnt1_ref = next(it), next(it)
        idx_v = [next(it) for _ in range(_NI)]
        rows_v = [next(it) for _ in range(_NB)]
        acc_sh = next(it)
        isem = [next(it) for _ in range(_NI)]
        gsem = [next(it) for _ in range(_NB)]
        ssem = [next(it) for _ in range(_NI)]
        if with_count:
            ones_v, cnt_sh, cbounce_v = next(it), next(it), next(it)

        c = lax.axis_index("c")
        s = lax.axis_index("s")
        wid = c * _NS + s
        r0 = s * rpt

        def zfill(i, carry):
            r = i // (D // 16)
            o = (i % (D // 16)) * 16
            rows_v[0][r, pl.ds(o, 16)] = jnp.zeros((16,), jnp.float32)
            return carry
        lax.fori_loop(0, _CHUNK * (D // 16), zfill, 0)
        nrep = rpt // _CHUNK
        for k in range(nrep):
            pltpu.sync_copy(rows_v[0],
                            acc_sh.at[pl.ds(r0 + k * _CHUNK, _CHUNK)])
        zr = rpt - nrep * _CHUNK
        if zr:
            pltpu.sync_copy(rows_v[0].at[pl.ds(0, zr)],
                            acc_sh.at[pl.ds(r0 + nrep * _CHUNK, zr)])
        if with_count:
            def zero16(i, carry):
                cbounce_v[pl.ds(i * 16, 16)] = jnp.zeros((16,), jnp.float32)
                return carry
            lax.fori_loop(0, rpt // 16, zero16, 0)
            pltpu.sync_copy(cbounce_v, cnt_sh.at[pl.ds(r0, rpt)])
            for i in range(_CHUNK // 16):
                ones_v[pl.ds(i * 16, 16)] = jnp.ones((16,), jnp.float32)
        if rrem:
            @pl.when(s == _NS - 1)
            def _():
                rr = _NS * rpt
                pltpu.sync_copy(rows_v[0].at[pl.ds(0, rrem)],
                                acc_sh.at[pl.ds(rr, rrem)])
                if with_count:
                    pltpu.sync_copy(cbounce_v.at[pl.ds(0, rrem)],
                                    cnt_sh.at[pl.ds(rr, rrem)])
        plsc.subcore_barrier()

        def echunk(g):
            return jnp.minimum(wid + g * NW, NCH - 1) * _CHUNK

        def idx_load(g, b):
            eb = echunk(g)
            pltpu.async_copy(edge_ref.at[:, pl.ds(eb, _CHUNK)],
                             idx_v[b], isem[b])

        def idx_wait(g, b):
            eb = echunk(g)
            pltpu.make_async_copy(edge_ref.at[:, pl.ds(eb, _CHUNK)],
                                  idx_v[b], isem[b]).wait()

        def gather_start(b2, b4):
            pltpu.async_copy(feat_ref.at[idx_v[b4].at[0]], rows_v[b2],
                             gsem[b2])

        def gather_wait(b2, b4):
            pltpu.make_async_copy(feat_ref.at[idx_v[b4].at[0]], rows_v[b2],
                                  gsem[b2]).wait()

        def scatter_start(b2, b4):
            pltpu.async_copy(rows_v[b2], acc_sh.at[idx_v[b4].at[1]],
                             ssem[b4], add=True)
            if with_count:
                pltpu.async_copy(ones_v, cnt_sh.at[idx_v[b4].at[1]],
                                 ssem[b4], add=True)

        def scatter_wait(b2, b4):
            pltpu.make_async_copy(rows_v[b2], acc_sh.at[idx_v[b4].at[1]],
                                  ssem[b4]).wait()
            if with_count:
                pltpu.make_async_copy(ones_v, cnt_sh.at[idx_v[b4].at[1]],
                                      ssem[b4]).wait()

        def steady(g, gb):
            b2, b4 = gb % _NB, gb % _NI
            p2, p4 = (gb - 1) % _NB, (gb - 1) % _NI
            idx_wait(g, b4)
            scatter_wait(b2, (gb - 2) % _NI)
            gather_start(b2, b4)
            gather_wait(p2, p4)
            scatter_start(p2, p4)
            idx_load(g + 2, (gb + 2) % _NI)

        idx_load(0, 0)
        idx_load(1, 1)
        idx_wait(0, 0)
        gather_start(0, 0)
        idx_load(2, 2)
        idx_wait(1, 1)
        gather_start(1, 1)
        gather_wait(0, 0)
        scatter_start(0, 0)
        idx_load(3, 3)

        n_steady = G - 2
        nq = n_steady // _NI

        def quad(q, carry):
            for r in range(_NI):
                steady(q * _NI + 2 + r, 2 + r)
            return carry

        lax.fori_loop(0, nq, quad, 0)
        for g in range(nq * _NI + 2, G):
            steady(g, g)

        gather_wait((G - 1) % _NB, (G - 1) % _NI)
        scatter_start((G - 1) % _NB, (G - 1) % _NI)
        idx_wait(G, G % _NI)
        idx_wait(G + 1, (G + 1) % _NI)
        scatter_wait(G % _NB, (G - 2) % _NI)
        if xrem:
            @pl.when(wid < xrem)
            def _():
                gather_start(G % _NB, G % _NI)
                gather_wait(G % _NB, G % _NI)
                scatter_start(G % _NB, G % _NI)
                scatter_wait(G % _NB, G % _NI)
        scatter_wait((G - 1) % _NB, (G - 1) % _NI)

        plsc.subcore_barrier()

        def flush(psum_ref, cnt_ref):
            pltpu.sync_copy(acc_sh.at[pl.ds(r0, rpt)],
                            psum_ref.at[pl.ds(r0, rpt)])
            if with_count:
                pltpu.sync_copy(cnt_sh.at[pl.ds(r0, rpt)], cbounce_v)
                pltpu.sync_copy(cbounce_v, cnt_ref.at[pl.ds(r0, rpt)])
            if rrem:
                @pl.when(s == _NS - 1)
                def _():
                    rr = _NS * rpt
                    pltpu.sync_copy(acc_sh.at[pl.ds(rr, rrem)],
                                    psum_ref.at[pl.ds(rr, rrem)])
                    if with_count:
                        pltpu.sync_copy(cnt_sh.at[pl.ds(rr, rrem)],
                                        cbounce_v.at[pl.ds(0, rrem)])
                        pltpu.sync_copy(cbounce_v.at[pl.ds(0, rrem)],
                                        cnt_ref.at[pl.ds(rr, rrem)])

        @pl.when(c == 0)
        def _():
            flush(psum0_ref, cnt0_ref if with_count else None)

        @pl.when(c == 1)
        def _():
            flush(psum1_ref, cnt1_ref if with_count else None)

    return pl.kernel(
        body,
        out_type=out_type,
        mesh=plsc.VectorSubcoreMesh(core_axis_name="c", subcore_axis_name="s"),
        scratch_types=scratch,
    )


def _tc_layer(N, D, H, relu, RB=5000):
    grid = (N // RB,)
    row_spec = pl.BlockSpec((RB, D), lambda i: (i, 0))
    w_spec = pl.BlockSpec((D, H), lambda i: (0, 0))
    b_spec = pl.BlockSpec((1, H), lambda i: (0, 0))

    def body(p0_ref, p1_ref, cc_ref, x_ref, wn_ref, ws_ref,
             b_ref, out_ref):
        cnt = cc_ref[:, 0:1] + cc_ref[:, 1:2]
        inv = 1.0 / jnp.maximum(cnt, 1.0)
        agg = (p0_ref[...] + p1_ref[...]) * inv
        acc = (jnp.dot(agg, wn_ref[...],
                       preferred_element_type=jnp.float32)
               + jnp.dot(x_ref[...], ws_ref[...],
                         preferred_element_type=jnp.float32)
               + b_ref[...])
        out_ref[...] = jnp.maximum(acc, 0.0) if relu else acc

    return pl.pallas_call(
        body,
        grid=grid,
        in_specs=[row_spec, row_spec,
                  pl.BlockSpec((RB, 2), lambda i: (i, 0)), row_spec,
                  w_spec, w_spec, b_spec],
        out_specs=pl.BlockSpec((RB, H), lambda i: (i, 0)),
        out_shape=jax.ShapeDtypeStruct((N, H), jnp.float32),
    )


def kernel(x, edge_index, W_neigh1, W_self1, b1, W_neigh2, W_self2, b2):
    N, D = x.shape
    H = W_neigh1.shape[1]
    E = edge_index.shape[1]

    p0, p1, c0, c1 = _make_sc_aggregate(N, D, E, with_count=True)(
        edge_index, x)
    cc = jnp.stack([c0, c1], axis=-1)
    h = _tc_layer(N, D, H, relu=True)(
        p0, p1, cc, x, W_neigh1, W_self1, b1.reshape(1, H))

    q0, q1 = _make_sc_aggregate(N, H, E, with_count=False)(edge_index, h)
    out = _tc_layer(N, H, H, relu=False)(
        q0, q1, cc, h, W_neigh2, W_self2, b2.reshape(1, H))
    return out
